# baseline (speedup 1.0000x reference)
import functools

import jax
import jax.numpy as jnp
from jax import lax
from jax.experimental import pallas as pl
from jax.experimental.pallas import tpu as pltpu
from jax.experimental.pallas import tpu_sc as plsc

B, L, D = 4096, 200, 64
V = 100000
N = B * L
NC, NS = 2, 16
NW = NC * NS
STEP = 128
TPW = N // NW
NSTEPS = TPW // STEP
NBUF = 4
NOBUF = 2


def _emb_body(tok_hbm, w_hbm, out_hbm, idx_v, rows, obuf,
              sem_idx, sem_in, sem_out):
    cid = lax.axis_index("c")
    sid = lax.axis_index("s")
    wid = sid * NC + cid
    obase = wid * TPW

    def fire_idx(t, b):
        for j in range(3):
            pltpu.async_copy(
                tok_hbm.at[pl.ds((obase + t * STEP) * 3 + j * STEP, STEP)],
                idx_v.at[b, j], sem_idx.at[b])

    def drain_idx(b):
        for j in range(3):
            pltpu.make_async_copy(tok_hbm.at[pl.ds(obase * 3, STEP)],
                                  idx_v.at[b, j], sem_idx.at[b]).wait()

    def fire_gather(b):
        for j in range(3):
            pltpu.async_copy(w_hbm.at[idx_v.at[b, j]],
                             rows.at[b, pl.ds(j * STEP, STEP)],
                             sem_in.at[b])

    def drain_gather(b):
        for j in range(3):
            pltpu.make_async_copy(w_hbm.at[idx_v.at[b, j]],
                                  rows.at[b, pl.ds(j * STEP, STEP)],
                                  sem_in.at[b]).wait()

    def accum(b, ob):
        @plsc.parallel_loop(0, STEP, unroll=4)
        def addrow(j):
            r = 3 * j
            for k in range(D // 16):
                sl = pl.ds(k * 16, 16)
                obuf[ob, j, sl] = (rows[b, r, sl] + rows[b, r + 1, sl]
                                   + rows[b, r + 2, sl])

    def fire_out(t, ob):
        pltpu.async_copy(obuf.at[ob],
                         out_hbm.at[pl.ds(obase + t * STEP, STEP)],
                         sem_out.at[ob])

    def drain_out(ob):
        pltpu.make_async_copy(obuf.at[ob],
                              out_hbm.at[pl.ds(obase, STEP)],
                              sem_out.at[ob]).wait()

    for t in range(NBUF):
        fire_idx(t, t)
    for t in range(2):
        drain_idx(t)
        fire_gather(t)

    def body4(u, carry):
        t0 = NBUF * u
        for b in range(NBUF):
            t = t0 + b
            ob = b % NOBUF

            @pl.when(t < NSTEPS - 2)
            def _():
                drain_idx((b + 2) % NBUF)
                fire_gather((b + 2) % NBUF)

            drain_gather(b)

            @pl.when(t < NSTEPS - NBUF)
            def _():
                fire_idx(t + NBUF, b)

            @pl.when(t >= NOBUF)
            def _():
                drain_out(ob)

            accum(b, ob)
            fire_out(t, ob)
        return carry

    lax.fori_loop(0, NSTEPS // NBUF, body4, 0)
    for ob in range(NOBUF):
        drain_out(ob)


@functools.partial(jax.jit, static_argnames=())
def _emb_call(tok3, W, ):
    mesh = plsc.VectorSubcoreMesh(core_axis_name="c", subcore_axis_name="s")
    return pl.kernel(
        _emb_body,
        out_type=jax.ShapeDtypeStruct((N, D), jnp.float32),
        mesh=mesh,
        scratch_types=[
            pltpu.VMEM((NBUF, 3, STEP), jnp.int32),
            pltpu.VMEM((NBUF, 3 * STEP, D), jnp.float32),
            pltpu.VMEM((NOBUF, STEP, D), jnp.float32),
            pltpu.SemaphoreType.DMA((NBUF,)),
            pltpu.SemaphoreType.DMA((NBUF,)),
            pltpu.SemaphoreType.DMA((NOBUF,)),
        ],
        compiler_params=pltpu.CompilerParams(use_tc_tiling_on_sc=False),
    )(tok3, W)


def kernel(factored_tokens, W0, W1, W2):
    W = jnp.concatenate([W0, W1, W2], axis=0)
    offs = jnp.array([0, V, 2 * V], dtype=jnp.int32)
    tok3 = (factored_tokens.reshape(N, 3).astype(jnp.int32)
            + offs[None, :]).reshape(N * 3)
    out = _emb_call(tok3, W)
    return out.reshape(B, L, D)


# speedup vs baseline: 2.7097x; 2.7097x over previous
import functools

import jax
import jax.numpy as jnp
from jax import lax
from jax.experimental import pallas as pl
from jax.experimental.pallas import tpu as pltpu
from jax.experimental.pallas import tpu_sc as plsc

B, L, D = 4096, 200, 64
V = 100000
N = B * L
NC, NS = 2, 16
NW = NC * NS
STEP = 128
TPW = N // NW
NSTEPS = TPW // STEP


def _emb_body(tok_hbm, w0_hbm, w1_hbm, w2_hbm, out_hbm,
              idx_v, rows, obuf, sem_idx, sem_in):
    cid = lax.axis_index("c")
    sid = lax.axis_index("s")
    wid = sid * NC + cid
    obase = wid * TPW
    ws = (w0_hbm, w1_hbm, w2_hbm)

    def step(t, carry):
        base = obase + t * STEP
        for j in range(3):
            pltpu.async_copy(tok_hbm.at[pl.ds(j * N + base, STEP)],
                             idx_v.at[j], sem_idx)
        for j in range(3):
            pltpu.make_async_copy(tok_hbm.at[pl.ds(j * N + base, STEP)],
                                  idx_v.at[j], sem_idx).wait()
        for j in range(3):
            pltpu.async_copy(ws[j].at[idx_v.at[j]], rows.at[j], sem_in)
        for j in range(3):
            pltpu.make_async_copy(ws[j].at[idx_v.at[j]], rows.at[j],
                                  sem_in).wait()

        @plsc.parallel_loop(0, STEP, unroll=4)
        def addrow(r):
            for k in range(D // 16):
                sl = pl.ds(k * 16, 16)
                obuf[r, sl] = rows[0, r, sl] + rows[1, r, sl] + rows[2, r, sl]

        pltpu.sync_copy(obuf, out_hbm.at[pl.ds(base, STEP)])
        return carry

    lax.fori_loop(0, NSTEPS, step, 0)


@jax.jit
def _emb_call(tokT, W0, W1, W2):
    mesh = plsc.VectorSubcoreMesh(core_axis_name="c", subcore_axis_name="s")
    return pl.kernel(
        _emb_body,
        out_type=jax.ShapeDtypeStruct((N, D), jnp.float32),
        mesh=mesh,
        scratch_types=[
            pltpu.VMEM((3, STEP), jnp.int32),
            pltpu.VMEM((3, STEP, D), jnp.float32),
            pltpu.VMEM((STEP, D), jnp.float32),
            pltpu.SemaphoreType.DMA,
            pltpu.SemaphoreType.DMA,
        ],
        compiler_params=pltpu.CompilerParams(use_tc_tiling_on_sc=False),
    )(tokT, W0, W1, W2)


def kernel(factored_tokens, W0, W1, W2):
    tokT = factored_tokens.reshape(N, 3).astype(jnp.int32).T.reshape(3 * N)
    out = _emb_call(tokT, W0, W1, W2)
    return out.reshape(B, L, D)


# speedup vs baseline: 3.6108x; 1.3325x over previous
import functools

import jax
import jax.numpy as jnp
from jax import lax
from jax.experimental import pallas as pl
from jax.experimental.pallas import tpu as pltpu
from jax.experimental.pallas import tpu_sc as plsc

B, L, D = 4096, 200, 64
V = 100000
N = B * L
NC, NS = 2, 16
NW = NC * NS
STEP = 128
TPW = N // NW
NSTEPS = TPW // STEP


def _emb_body(tok_hbm, w0_hbm, w1_hbm, w2_hbm, out_hbm,
              idx_v, rows, obuf, sem_idx, sem_in, sem_out):
    cid = lax.axis_index("c")
    sid = lax.axis_index("s")
    wid = sid * NC + cid
    obase = wid * TPW
    ws = (w0_hbm, w1_hbm, w2_hbm)

    def fire(t, b):
        base = obase + t * STEP
        for j in range(3):
            pltpu.async_copy(tok_hbm.at[pl.ds(j * N + base, STEP)],
                             idx_v.at[b, j], sem_idx.at[b])
        for j in range(3):
            pltpu.make_async_copy(tok_hbm.at[pl.ds(j * N + base, STEP)],
                                  idx_v.at[b, j], sem_idx.at[b]).wait()
        for j in range(3):
            pltpu.async_copy(ws[j].at[idx_v.at[b, j]], rows.at[b, j],
                             sem_in.at[b])

    def drain(b):
        for j in range(3):
            pltpu.make_async_copy(ws[j].at[idx_v.at[b, j]], rows.at[b, j],
                                  sem_in.at[b]).wait()

    def accum(b):
        @plsc.parallel_loop(0, STEP, unroll=4)
        def addrow(r):
            for k in range(D // 16):
                sl = pl.ds(k * 16, 16)
                obuf[b, r, sl] = (rows[b, 0, r, sl] + rows[b, 1, r, sl]
                                  + rows[b, 2, r, sl])

    def fire_out(t, b):
        pltpu.async_copy(obuf.at[b],
                         out_hbm.at[pl.ds(obase + t * STEP, STEP)],
                         sem_out.at[b])

    def drain_out(b):
        pltpu.make_async_copy(obuf.at[b],
                              out_hbm.at[pl.ds(obase, STEP)],
                              sem_out.at[b]).wait()

    fire(0, 0)

    def body2(u, carry):
        t0 = 2 * u
        for b in range(2):
            t = t0 + b

            @pl.when(t < NSTEPS - 1)
            def _():
                fire(t + 1, 1 - b)

            drain(b)

            @pl.when(t >= 2)
            def _():
                drain_out(b)

            accum(b)
            fire_out(t, b)
        return carry

    lax.fori_loop(0, NSTEPS // 2, body2, 0)
    drain_out(0)
    drain_out(1)


@jax.jit
def _emb_call(tokT, W0, W1, W2):
    mesh = plsc.VectorSubcoreMesh(core_axis_name="c", subcore_axis_name="s")
    return pl.kernel(
        _emb_body,
        out_type=jax.ShapeDtypeStruct((N, D), jnp.float32),
        mesh=mesh,
        scratch_types=[
            pltpu.VMEM((2, 3, STEP), jnp.int32),
            pltpu.VMEM((2, 3, STEP, D), jnp.float32),
            pltpu.VMEM((2, STEP, D), jnp.float32),
            pltpu.SemaphoreType.DMA((2,)),
            pltpu.SemaphoreType.DMA((2,)),
            pltpu.SemaphoreType.DMA((2,)),
        ],
        compiler_params=pltpu.CompilerParams(use_tc_tiling_on_sc=False),
    )(tokT, W0, W1, W2)


def kernel(factored_tokens, W0, W1, W2):
    tokT = factored_tokens.reshape(N, 3).astype(jnp.int32).T.reshape(3 * N)
    out = _emb_call(tokT, W0, W1, W2)
    return out.reshape(B, L, D)
